# baseline (speedup 1.0000x reference)
import jax
import jax.numpy as jnp
from jax import lax
from jax.experimental import pallas as pl
from jax.experimental.pallas import tpu as pltpu
from jax.experimental.pallas import tpu_sc as plsc

E_NEW = 65536
NUM_CORES = 2
NUM_SUBCORES = 16
LANES = 16
CHUNK = 4000
UNROLL = 5
SUNROLL = 10


NBUF = 3


def _unpool_body(feat_hbm, oidx_hbm, l_hbm, r_hbm, ne_hbm, nel_hbm, ner_hbm,
                 out_hbm, out_v, feat_v0, feat_v1, feat_v2,
                 oidx_v0, oidx_v1, oidx_v2,
                 l_v, r_v, ne_v, nel_v, ner_v,
                 sem_a, sem_b, sem_c, sem_out, *, B, C, E_old, U):
    n_chunks = E_old // CHUNK
    sems = (sem_a, sem_b, sem_c)
    feat_bufs = (feat_v0, feat_v1, feat_v2)
    oidx_bufs = (oidx_v0, oidx_v1, oidx_v2)

    cid = lax.axis_index("c")
    sid = lax.axis_index("s")
    wid = cid * NUM_SUBCORES + sid
    nw = NUM_CORES * NUM_SUBCORES
    workers_per_b = nw // B
    rows_per_worker = C // workers_per_b
    b = wid // workers_per_b
    c0 = (wid % workers_per_b) * rows_per_worker
    zeros = jnp.zeros((LANES,), jnp.float32)

    def zbody(i, _):
        for u in range(8):
            out_v[pl.ds(i * 8 * LANES + u * LANES, LANES)] = zeros
        return _

    lax.fori_loop(0, E_NEW // (8 * LANES), zbody, None)

    pltpu.sync_copy(l_hbm.at[pl.ds(b * U, U)], l_v)
    pltpu.sync_copy(r_hbm.at[pl.ds(b * U, U)], r_v)
    pltpu.sync_copy(ne_hbm.at[pl.ds(b * U, U)], ne_v)
    pltpu.sync_copy(nel_hbm.at[pl.ds(b * U, U)], nel_v)
    pltpu.sync_copy(ner_hbm.at[pl.ds(b * U, U)], ner_v)

    def issue_chunk(c, k):
        slot = k % NBUF
        feat_base = (b * C + c) * E_old
        cp_o = pltpu.async_copy(
            oidx_hbm.at[pl.ds(b * E_old + k * CHUNK, CHUNK)],
            oidx_bufs[slot], sems[slot])
        cp_f = pltpu.async_copy(
            feat_hbm.at[pl.ds(feat_base + k * CHUNK, CHUNK)],
            feat_bufs[slot], sems[slot])
        return cp_o, cp_f

    def wait_slot(slot):
        pltpu.make_async_copy(oidx_hbm.at[pl.ds(0, CHUNK)],
                              oidx_bufs[slot], sems[slot]).wait()
        pltpu.make_async_copy(feat_hbm.at[pl.ds(0, CHUNK)],
                              feat_bufs[slot], sems[slot]).wait()

    for k in range(NBUF):
        issue_chunk(c0, k)

    def row(ci, _):
        c = c0 + ci

        @pl.when(ci > 0)
        def _():
            pltpu.make_async_copy(out_v, out_hbm.at[pl.ds(0, E_NEW)],
                                  sem_out).wait()

        pending = {}
        for k in range(n_chunks):
            slot = k % NBUF
            if k < NBUF:
                wait_slot(slot)
            else:
                cp_o, cp_f = pending.pop(k)
                cp_o.wait()
                cp_f.wait()
            ob = oidx_bufs[slot]
            fb = feat_bufs[slot]

            def scat(i, _):
                for u in range(SUNROLL):
                    sl = pl.ds(i * SUNROLL * LANES + u * LANES, LANES)
                    plsc.store_scatter(out_v, [ob[sl]], fb[sl])
                return _

            lax.fori_loop(0, CHUNK // (SUNROLL * LANES), scat, None)
            if k + NBUF < n_chunks:
                pending[k + NBUF] = issue_chunk(c, k + NBUF)

        @pl.when(ci + 1 < rows_per_worker)
        def _():
            for k in range(NBUF):
                issue_chunk(c + 1, k)

        def child(i, _):
            for u in range(UNROLL):
                sl = pl.ds(i * UNROLL * LANES + u * LANES, LANES)
                lf = plsc.load_gather(out_v, [l_v[sl]])
                rf = plsc.load_gather(out_v, [r_v[sl]])
                plsc.store_scatter(out_v, [nel_v[sl]], lf)
                plsc.store_scatter(out_v, [ner_v[sl]], rf)
                plsc.store_scatter(out_v, [ne_v[sl]],
                                   (lf + rf) * jnp.float32(0.5))
            return _

        lax.fori_loop(0, U // (UNROLL * LANES), child, None)

        pltpu.async_copy(out_v, out_hbm.at[pl.ds((b * C + c) * E_NEW, E_NEW)],
                         sem_out)
        return _

    lax.fori_loop(0, rows_per_worker, row, None)
    pltpu.make_async_copy(out_v, out_hbm.at[pl.ds(0, E_NEW)], sem_out).wait()


def kernel(features, old_indices, left_idx, right_idx, new_e_idx,
           new_e_left_idx, new_e_right_idx):
    B, C, E_old = features.shape
    U = left_idx.shape[1]

    mesh = plsc.VectorSubcoreMesh(core_axis_name="c", subcore_axis_name="s",
                                  num_cores=NUM_CORES,
                                  num_subcores=NUM_SUBCORES)

    def body(*refs):
        _unpool_body(*refs, B=B, C=C, E_old=E_old, U=U)

    run = pl.kernel(
        body,
        out_type=jax.ShapeDtypeStruct((B * C * E_NEW,), jnp.float32),
        mesh=mesh,
        scratch_types=[
            pltpu.VMEM((E_NEW,), jnp.float32),
            pltpu.VMEM((CHUNK,), jnp.float32),
            pltpu.VMEM((CHUNK,), jnp.float32),
            pltpu.VMEM((CHUNK,), jnp.float32),
            pltpu.VMEM((CHUNK,), jnp.int32),
            pltpu.VMEM((CHUNK,), jnp.int32),
            pltpu.VMEM((CHUNK,), jnp.int32),
            pltpu.VMEM((U,), jnp.int32),
            pltpu.VMEM((U,), jnp.int32),
            pltpu.VMEM((U,), jnp.int32),
            pltpu.VMEM((U,), jnp.int32),
            pltpu.VMEM((U,), jnp.int32),
            pltpu.SemaphoreType.DMA,
            pltpu.SemaphoreType.DMA,
            pltpu.SemaphoreType.DMA,
            pltpu.SemaphoreType.DMA,
        ],
        compiler_params=pltpu.CompilerParams(needs_layout_passes=False),
    )
    out_flat = run(features.reshape(-1), old_indices.reshape(-1),
                   left_idx.reshape(-1), right_idx.reshape(-1),
                   new_e_idx.reshape(-1), new_e_left_idx.reshape(-1),
                   new_e_right_idx.reshape(-1))
    return out_flat.reshape(B, C, E_NEW)


# speedup vs baseline: 1.3235x; 1.3235x over previous
import jax
import jax.numpy as jnp
from jax import lax
from jax.experimental import pallas as pl
from jax.experimental.pallas import tpu as pltpu
from jax.experimental.pallas import tpu_sc as plsc

E_NEW = 65536
NUM_CORES = 2
NUM_SUBCORES = 16
LANES = 16
CHUNK = 4000
UNROLL = 5
SUNROLL = 10


NBUF = 3


def _unpool_body(feat_hbm, oidx_hbm, l_hbm, r_hbm, ne_hbm, nel_hbm, ner_hbm,
                 out_hbm, out_v, feat_v0, feat_v1, feat_v2,
                 oidx_v0, oidx_v1, oidx_v2,
                 l_v, r_v, ne_v, nel_v, ner_v, ix0, ix1, ix2, ix3,
                 sem_a, sem_b, sem_c, sem_out, *, B, C, E_old, U):
    n_chunks = E_old // CHUNK
    sems = (sem_a, sem_b, sem_c)
    feat_bufs = (feat_v0, feat_v1, feat_v2)
    oidx_bufs = (oidx_v0, oidx_v1, oidx_v2)

    cid = lax.axis_index("c")
    sid = lax.axis_index("s")
    wid = cid * NUM_SUBCORES + sid
    nw = NUM_CORES * NUM_SUBCORES
    workers_per_b = nw // B
    rows_per_worker = C // workers_per_b
    b = wid // workers_per_b
    c0 = (wid % workers_per_b) * rows_per_worker
    zeros = jnp.zeros((LANES,), jnp.float32)
    iota = lax.iota(jnp.int32, LANES)
    idx_bufs = (ix0, ix1, ix2, ix3)

    def zbody(i, _):
        for u in range(8):
            dv = (i * 8 + u) * LANES + iota
            plsc.store_scatter(out_v, [lax.shift_right_logical(dv, 7),
                                       dv & 127], zeros)
        return _

    lax.fori_loop(0, E_NEW // (8 * LANES), zbody, None)

    pltpu.sync_copy(l_hbm.at[pl.ds(b * U, U)], l_v)
    pltpu.sync_copy(r_hbm.at[pl.ds(b * U, U)], r_v)
    pltpu.sync_copy(ne_hbm.at[pl.ds(b * U, U)], ne_v)
    pltpu.sync_copy(nel_hbm.at[pl.ds(b * U, U)], nel_v)
    pltpu.sync_copy(ner_hbm.at[pl.ds(b * U, U)], ner_v)

    def issue_chunk(c, k):
        slot = k % NBUF
        feat_base = (b * C + c) * E_old
        cp_o = pltpu.async_copy(
            oidx_hbm.at[pl.ds(b * E_old + k * CHUNK, CHUNK)],
            oidx_bufs[slot], sems[slot])
        cp_f = pltpu.async_copy(
            feat_hbm.at[pl.ds(feat_base + k * CHUNK, CHUNK)],
            feat_bufs[slot], sems[slot])
        return cp_o, cp_f

    def wait_slot(slot):
        pltpu.make_async_copy(oidx_hbm.at[pl.ds(0, CHUNK)],
                              oidx_bufs[slot], sems[slot]).wait()
        pltpu.make_async_copy(feat_hbm.at[pl.ds(0, CHUNK)],
                              feat_bufs[slot], sems[slot]).wait()

    for k in range(NBUF):
        issue_chunk(c0, k)

    def row(ci, _):
        c = c0 + ci

        @pl.when(ci > 0)
        def _():
            for q in range(4):
                pltpu.make_async_copy(out_v.at[pl.ds(q * 128, 128)],
                                      out_hbm.at[idx_bufs[q]],
                                      sem_out).wait()

        pending = {}
        for k in range(n_chunks):
            slot = k % NBUF
            if k < NBUF:
                wait_slot(slot)
            else:
                cp_o, cp_f = pending.pop(k)
                cp_o.wait()
                cp_f.wait()
            ob = oidx_bufs[slot]
            fb = feat_bufs[slot]

            def scat(i, _):
                for u in range(SUNROLL):
                    sl = pl.ds(i * SUNROLL * LANES + u * LANES, LANES)
                    dv = ob[sl]
                    plsc.store_scatter(out_v,
                                       [lax.shift_right_logical(dv, 7),
                                        dv & 127], fb[sl])
                return _

            lax.fori_loop(0, CHUNK // (SUNROLL * LANES), scat, None)
            if k + NBUF < n_chunks:
                pending[k + NBUF] = issue_chunk(c, k + NBUF)

        @pl.when(ci + 1 < rows_per_worker)
        def _():
            for k in range(NBUF):
                issue_chunk(c + 1, k)

        def child(i, _):
            for u in range(UNROLL):
                sl = pl.ds(i * UNROLL * LANES + u * LANES, LANES)
                lv = l_v[sl]
                rv = r_v[sl]
                lf = plsc.load_gather(out_v, [lax.shift_right_logical(lv, 7),
                                              lv & 127])
                rf = plsc.load_gather(out_v, [lax.shift_right_logical(rv, 7),
                                              rv & 127])
                nlv = nel_v[sl]
                nrv = ner_v[sl]
                nev = ne_v[sl]
                plsc.store_scatter(out_v, [lax.shift_right_logical(nlv, 7),
                                           nlv & 127], lf)
                plsc.store_scatter(out_v, [lax.shift_right_logical(nrv, 7),
                                           nrv & 127], rf)
                plsc.store_scatter(out_v, [lax.shift_right_logical(nev, 7),
                                           nev & 127],
                                   (lf + rf) * jnp.float32(0.5))
            return _

        lax.fori_loop(0, U // (UNROLL * LANES), child, None)

        nb = ((b * (C // 8) + c // 8) * (E_NEW // 128)) * 8 + (c % 8)
        for q in range(4):
            for u in range(8):
                et = q * 128 + u * LANES
                idx_bufs[q][pl.ds(u * LANES, LANES)] = nb + (et + iota) * 8
        for q in range(4):
            pltpu.async_copy(out_v.at[pl.ds(q * 128, 128)],
                             out_hbm.at[idx_bufs[q]], sem_out)
        return _

    lax.fori_loop(0, rows_per_worker, row, None)
    for q in range(4):
        pltpu.make_async_copy(out_v.at[pl.ds(q * 128, 128)],
                              out_hbm.at[idx_bufs[q]], sem_out).wait()


def kernel(features, old_indices, left_idx, right_idx, new_e_idx,
           new_e_left_idx, new_e_right_idx):
    B, C, E_old = features.shape
    U = left_idx.shape[1]

    mesh = plsc.VectorSubcoreMesh(core_axis_name="c", subcore_axis_name="s",
                                  num_cores=NUM_CORES,
                                  num_subcores=NUM_SUBCORES)

    def body(*refs):
        _unpool_body(*refs, B=B, C=C, E_old=E_old, U=U)

    run = pl.kernel(
        body,
        out_type=jax.ShapeDtypeStruct((B * C * (E_NEW // 128), 128),
                                      jnp.float32),
        mesh=mesh,
        scratch_types=[
            pltpu.VMEM((E_NEW // 128, 128), jnp.float32),
            pltpu.VMEM((CHUNK,), jnp.float32),
            pltpu.VMEM((CHUNK,), jnp.float32),
            pltpu.VMEM((CHUNK,), jnp.float32),
            pltpu.VMEM((CHUNK,), jnp.int32),
            pltpu.VMEM((CHUNK,), jnp.int32),
            pltpu.VMEM((CHUNK,), jnp.int32),
            pltpu.VMEM((U,), jnp.int32),
            pltpu.VMEM((U,), jnp.int32),
            pltpu.VMEM((U,), jnp.int32),
            pltpu.VMEM((U,), jnp.int32),
            pltpu.VMEM((U,), jnp.int32),
            pltpu.VMEM((128,), jnp.int32),
            pltpu.VMEM((128,), jnp.int32),
            pltpu.VMEM((128,), jnp.int32),
            pltpu.VMEM((128,), jnp.int32),
            pltpu.SemaphoreType.DMA,
            pltpu.SemaphoreType.DMA,
            pltpu.SemaphoreType.DMA,
            pltpu.SemaphoreType.DMA,
        ],
        compiler_params=pltpu.CompilerParams(needs_layout_passes=False),
    )
    out2 = run(features.reshape(-1), old_indices.reshape(-1),
               left_idx.reshape(-1), right_idx.reshape(-1),
               new_e_idx.reshape(-1), new_e_left_idx.reshape(-1),
               new_e_right_idx.reshape(-1))
    out5 = out2.reshape(B, C // 8, E_NEW // 128, 8, 128)
    return out5.transpose(0, 1, 3, 2, 4).reshape(B, C, E_NEW)
